# baseline (speedup 1.0000x reference)
import functools

import jax
import jax.numpy as jnp
from jax import lax
from jax.experimental import pallas as pl
from jax.experimental.pallas import tpu as pltpu
from jax.experimental.pallas import tpu_sc as plsc

BATCH = 16384
EMB_DIM = 32
NC = 2
NS = 16
NW = NC * NS
BPW = BATCH // NW

CBLKS = (1000000 + 127) // 128
FLAT_WORDS = 4 * CBLKS * 1024


def _distmult_body(h_hbm, r_hbm, t_hbm, ent_t_hbm, rel_t_hbm, out_hbm,
                   hidx_v, ridx_v, tidx_v, eh_v, er_v, et_v, out_v, sem):
    wid = lax.axis_index("s") * NC + lax.axis_index("c")
    base = wid * BPW


    pltpu.sync_copy(h_hbm.at[pl.ds(base, BPW)], hidx_v)
    pltpu.sync_copy(r_hbm.at[pl.ds(base, BPW)], ridx_v)
    pltpu.sync_copy(t_hbm.at[pl.ds(base, BPW)], tidx_v)

    copies = []
    for d in range(EMB_DIM):
        copies.append(pltpu.async_copy(
            ent_t_hbm.at[d].at[hidx_v], eh_v.at[d], sem))
        copies.append(pltpu.async_copy(
            rel_t_hbm.at[d].at[ridx_v], er_v.at[d], sem))
        copies.append(pltpu.async_copy(
            ent_t_hbm.at[d].at[tidx_v], et_v.at[d], sem))
    for c in copies:
        c.wait()

    def group(g, _):
        s = g * 16
        acc = jnp.zeros((16,), jnp.float32)
        for d in range(EMB_DIM):
            acc = acc + (eh_v[d, pl.ds(s, 16)]
                         * er_v[d, pl.ds(s, 16)]
                         * et_v[d, pl.ds(s, 16)])
        out_v[pl.ds(s, 16)] = acc
        return 0

    lax.fori_loop(0, BPW // 16, group, 0)

    pltpu.sync_copy(out_v, out_hbm.at[pl.ds(base, BPW)])


@jax.jit
def _distmult(hs, rs, ts, ent_t, rel_t):
    mesh = plsc.VectorSubcoreMesh(core_axis_name="c", subcore_axis_name="s")
    kern = functools.partial(
        pl.kernel,
        mesh=mesh,
        compiler_params=pltpu.CompilerParams(
            needs_layout_passes=False, use_tc_tiling_on_sc=False),
        out_type=jax.ShapeDtypeStruct((BATCH,), jnp.float32),
        scratch_types=[
            pltpu.VMEM((BPW,), jnp.int32),
            pltpu.VMEM((BPW,), jnp.int32),
            pltpu.VMEM((BPW,), jnp.int32),
            pltpu.VMEM((EMB_DIM, BPW), jnp.float32),
            pltpu.VMEM((EMB_DIM, BPW), jnp.float32),
            pltpu.VMEM((EMB_DIM, BPW), jnp.float32),
            pltpu.VMEM((BPW,), jnp.float32),
            pltpu.SemaphoreType.DMA,
        ],
    )(_distmult_body)
    return kern(hs, rs, ts, ent_t, rel_t)


def kernel(batch, ent_embs, rel_embs):
    hs = batch[:, 0]
    rs = batch[:, 1]
    ts = batch[:, 2]
    return _distmult(hs, rs, ts, ent_embs.T, rel_embs.T)


# speedup vs baseline: 13.5304x; 13.5304x over previous
import functools

import jax
import jax.numpy as jnp
from jax import lax
from jax.experimental import pallas as pl
from jax.experimental.pallas import tpu as pltpu
from jax.experimental.pallas import tpu_sc as plsc

BATCH = 16384
EMB_DIM = 32
NC = 2
NS = 16
NW = NC * NS
BPW = BATCH // NW
GRP = BPW // 16


def _distmult_body(h_hbm, r_hbm, t_hbm, ent_t_hbm, rel_t_hbm, out_hbm,
                   hidx_v, ridx_v, tidx_v, win_v, stage_v, out_v, sem):
    wid = lax.axis_index("s") * NC + lax.axis_index("c")
    base = wid * BPW

    pltpu.sync_copy(h_hbm.at[pl.ds(base, BPW)], hidx_v)
    pltpu.sync_copy(r_hbm.at[pl.ds(base, BPW)], ridx_v)
    pltpu.sync_copy(t_hbm.at[pl.ds(base, BPW)], tidx_v)

    lanes = lax.iota(jnp.int32, 16)

    def make_phase(idx_v, tab_hbm, tau):
        def phase(g, _):
            jvec = idx_v[pl.ds(g * 16, 16)]
            for i in range(16):
                jb128 = (jvec[i] >> 7) * 128
                pltpu.make_async_copy(
                    tab_hbm.at[:, pl.ds(pl.multiple_of(jb128, 128), 128)],
                    win_v.at[i], sem,
                ).start()
            for i in range(16):
                pltpu.make_async_copy(
                    tab_hbm.at[:, pl.ds(0, 128)], win_v.at[i], sem,
                ).wait()
            mvec = jvec & 127
            for i in range(16):
                mv = jnp.full((16,), mvec[i], jnp.int32)
                sv = jnp.full((16,), i, jnp.int32)
                p0 = plsc.load_gather(win_v, [sv, lanes, mv])
                p1 = plsc.load_gather(win_v, [sv, lanes + 16, mv])
                off = (tau * BPW + g * 16 + i) * EMB_DIM
                stage_v[pl.ds(off, 16)] = p0
                stage_v[pl.ds(off + 16, 16)] = p1
            return 0
        return phase

    lax.fori_loop(0, GRP, make_phase(hidx_v, ent_t_hbm, 0), 0)
    lax.fori_loop(0, GRP, make_phase(ridx_v, rel_t_hbm, 1), 0)
    lax.fori_loop(0, GRP, make_phase(tidx_v, ent_t_hbm, 2), 0)

    def group(g, _):
        s = g * 16
        acc = jnp.zeros((16,), jnp.float32)
        for i in range(16):
            r = s + i
            h0 = stage_v[pl.ds(r * EMB_DIM, 16)]
            h1 = stage_v[pl.ds(r * EMB_DIM + 16, 16)]
            r0 = stage_v[pl.ds((BPW + r) * EMB_DIM, 16)]
            r1 = stage_v[pl.ds((BPW + r) * EMB_DIM + 16, 16)]
            t0 = stage_v[pl.ds((2 * BPW + r) * EMB_DIM, 16)]
            t1 = stage_v[pl.ds((2 * BPW + r) * EMB_DIM + 16, 16)]
            half = h0 * r0 * t0 + h1 * r1 * t1
            acc = jnp.where(lanes == i, jnp.sum(half), acc)
        out_v[pl.ds(s, 16)] = acc
        return 0

    lax.fori_loop(0, GRP, group, 0)

    pltpu.sync_copy(out_v, out_hbm.at[pl.ds(base, BPW)])


@jax.jit
def _distmult(hs, rs, ts, ent_t, rel_t):
    mesh = plsc.VectorSubcoreMesh(core_axis_name="c", subcore_axis_name="s")
    kern = functools.partial(
        pl.kernel,
        mesh=mesh,
        compiler_params=pltpu.CompilerParams(
            needs_layout_passes=False, use_tc_tiling_on_sc=True),
        out_type=jax.ShapeDtypeStruct((BATCH,), jnp.float32),
        scratch_types=[
            pltpu.VMEM((BPW,), jnp.int32),
            pltpu.VMEM((BPW,), jnp.int32),
            pltpu.VMEM((BPW,), jnp.int32),
            pltpu.VMEM((16, EMB_DIM, 128), jnp.float32),
            pltpu.VMEM((3 * BPW * EMB_DIM,), jnp.float32),
            pltpu.VMEM((BPW,), jnp.float32),
            pltpu.SemaphoreType.DMA,
        ],
    )(_distmult_body)
    return kern(hs, rs, ts, ent_t, rel_t)


def kernel(batch, ent_embs, rel_embs):
    hs = batch[:, 0]
    rs = batch[:, 1]
    ts = batch[:, 2]
    return _distmult(hs, rs, ts, ent_embs.T, rel_embs.T)
